# baseline (device time: 26766 ns/iter reference)
import pathlib

import jax
import jax.numpy as jnp
from jax import lax
from jax.experimental import pallas as pl
from jax.experimental.pallas import tpu as pltpu

SCALE = 64 ** -0.5

_MODE_FILE = pathlib.Path(__file__).parent / "bench_mode.txt"
MODE = _MODE_FILE.read_text().strip() if _MODE_FILE.exists() else "full"

N_HALF = 8


def _body(q_ref, k_ref, v_ref, o_ref, ko_ref, vo_ref, l_ref,
          sa_sems, ra_sems, sb_sems, rb_sems):
    my_x = lax.axis_index("x")
    my_y = lax.axis_index("y")
    nbr_y = (my_x, 1 - my_y)
    nbr_x = (1 - my_x, my_y)

    comm = MODE in ("full", "comm")
    compute = MODE not in ("comm", "raw", "noexit")

    if MODE == "raw":
        o_ref[...] = jnp.zeros_like(o_ref)
        return

    barrier_sem = pltpu.get_barrier_semaphore()
    for nbr in (nbr_y, nbr_x):
        pl.semaphore_signal(barrier_sem, inc=1, device_id=nbr,
                            device_id_type=pl.DeviceIdType.MESH)
    pl.semaphore_wait(barrier_sem, 2)

    base_a = my_x * N_HALF
    base_b = (1 - my_x) * N_HALF

    if MODE in ("compute", "none"):
        ko_ref, vo_ref = k_ref, v_ref

    rdma_a = []
    if comm:
        for h in range(N_HALF):
            hd = base_a + h
            rk = pltpu.make_async_remote_copy(
                src_ref=k_ref.at[hd], dst_ref=ko_ref.at[hd],
                send_sem=sa_sems.at[0, h], recv_sem=ra_sems.at[0, h],
                device_id=nbr_y, device_id_type=pl.DeviceIdType.MESH)
            rv = pltpu.make_async_remote_copy(
                src_ref=v_ref.at[hd], dst_ref=vo_ref.at[hd],
                send_sem=sa_sems.at[1, h], recv_sem=ra_sems.at[1, h],
                device_id=nbr_y, device_id_type=pl.DeviceIdType.MESH)
            rk.start()
            rv.start()
            rdma_a.append((rk, rv))

    def local_head(i):
        qt = q_ref[i]
        st = lax.dot_general(k_ref[i], qt, (((0,), (0,)), ((), ())),
                             preferred_element_type=jnp.float32)
        p = jnp.exp(st)
        l_ref[i] = jnp.sum(p, axis=0)
        acc = lax.dot_general(v_ref[i], p.astype(jnp.bfloat16),
                              (((1,), (0,)), ((), ())),
                              preferred_element_type=jnp.float32)
        o_ref[i] = acc.astype(jnp.bfloat16)

    def remote_head(i):
        qt = q_ref[i]
        st = lax.dot_general(ko_ref[i], qt, (((0,), (0,)), ((), ())),
                             preferred_element_type=jnp.float32)
        p = jnp.exp(st)
        l = l_ref[i] + jnp.sum(p, axis=0)
        acc = (o_ref[i].astype(jnp.float32) +
               lax.dot_general(vo_ref[i], p.astype(jnp.bfloat16),
                               (((1,), (0,)), ((), ())),
                               preferred_element_type=jnp.float32))
        o_ref[i] = (acc / l[None, :]).astype(jnp.bfloat16)

    rdma_b = []
    for h in range(N_HALF):
        if compute:
            local_head(base_a + h)
            local_head(base_b + h)
        if comm:
            rk, rv = rdma_a[h]
            rk.wait()
            rv.wait()
            hd = base_a + h
            fk = pltpu.make_async_remote_copy(
                src_ref=ko_ref.at[hd], dst_ref=ko_ref.at[hd],
                send_sem=sb_sems.at[0, h], recv_sem=rb_sems.at[0, h],
                device_id=nbr_x, device_id_type=pl.DeviceIdType.MESH)
            fv = pltpu.make_async_remote_copy(
                src_ref=vo_ref.at[hd], dst_ref=vo_ref.at[hd],
                send_sem=sb_sems.at[1, h], recv_sem=rb_sems.at[1, h],
                device_id=nbr_x, device_id_type=pl.DeviceIdType.MESH)
            fk.start()
            fv.start()
            rdma_b.append((fk, fv))
        if compute:
            remote_head(base_a + h)

    for h in range(N_HALF):
        if comm:
            fk, fv = rdma_b[h]
            fk.wait()
            fv.wait()
        if compute:
            remote_head(base_b + h)

    if MODE in ("comm", "noexit"):
        o_ref[...] = jnp.zeros_like(o_ref)


def kernel(Q, K, V):
    b, s, h, d = Q.shape

    def prep(a):
        return a.astype(jnp.bfloat16).transpose(0, 2, 3, 1).reshape(b * h, d, s)

    out = pl.pallas_call(
        _body,
        out_shape=jax.ShapeDtypeStruct((b * h, d, s), jnp.bfloat16),
        in_specs=[pl.BlockSpec(memory_space=pltpu.VMEM)] * 3,
        out_specs=pl.BlockSpec(memory_space=pltpu.VMEM),
        scratch_shapes=[
            pltpu.VMEM((b * h, d, s), jnp.bfloat16),
            pltpu.VMEM((b * h, d, s), jnp.bfloat16),
            pltpu.VMEM((b * h, s), jnp.float32),
            pltpu.SemaphoreType.DMA((2, N_HALF)),
            pltpu.SemaphoreType.DMA((2, N_HALF)),
            pltpu.SemaphoreType.DMA((2, N_HALF)),
            pltpu.SemaphoreType.DMA((2, N_HALF)),
        ],
        compiler_params=(pltpu.CompilerParams() if MODE == "raw"
                         else pltpu.CompilerParams(collective_id=0)),
    )(prep(Q * SCALE), prep(K), prep(V))
    return out.reshape(b, h, d, s).transpose(0, 3, 1, 2).astype(jnp.float32)


# device time: 26152 ns/iter; 1.0235x vs baseline; 1.0235x over previous
import pathlib

import jax
import jax.numpy as jnp
from jax import lax
from jax.experimental import pallas as pl
from jax.experimental.pallas import tpu as pltpu

SCALE = 64 ** -0.5

_MODE_FILE = pathlib.Path(__file__).parent / "bench_mode.txt"
MODE = _MODE_FILE.read_text().strip() if _MODE_FILE.exists() else "full"

N_HALF = 8


def _body(q_ref, k_ref, v_ref, o_ref, ko_ref, vo_ref, l_ref,
          sa_sems, ra_sems, sb_sems, rb_sems):
    my_x = lax.axis_index("x")
    my_y = lax.axis_index("y")
    nbr_y = (my_x, 1 - my_y)
    nbr_x = (1 - my_x, my_y)

    comm = MODE in ("full", "comm")
    compute = MODE not in ("comm", "raw", "noexit")

    if MODE == "raw":
        o_ref[...] = jnp.zeros_like(o_ref)
        return

    barrier_sem = pltpu.get_barrier_semaphore()
    for nbr in (nbr_y, nbr_x):
        pl.semaphore_signal(barrier_sem, inc=1, device_id=nbr,
                            device_id_type=pl.DeviceIdType.MESH)
    pl.semaphore_wait(barrier_sem, 2)

    base_a = my_x * N_HALF
    base_b = (1 - my_x) * N_HALF

    if MODE in ("compute", "none"):
        ko_ref, vo_ref = k_ref, v_ref

    rdma_a = []
    if comm:
        for h in range(N_HALF):
            hd = base_a + h
            rk = pltpu.make_async_remote_copy(
                src_ref=k_ref.at[hd], dst_ref=ko_ref.at[hd],
                send_sem=sa_sems.at[0, h], recv_sem=ra_sems.at[0, h],
                device_id=nbr_y, device_id_type=pl.DeviceIdType.MESH)
            rv = pltpu.make_async_remote_copy(
                src_ref=v_ref.at[hd], dst_ref=vo_ref.at[hd],
                send_sem=sa_sems.at[1, h], recv_sem=ra_sems.at[1, h],
                device_id=nbr_y, device_id_type=pl.DeviceIdType.MESH)
            rk.start()
            rv.start()
            rdma_a.append((rk, rv))

    def local_head(i):
        qt = q_ref[i]
        st = lax.dot_general(k_ref[i], qt, (((0,), (0,)), ((), ())),
                             preferred_element_type=jnp.float32)
        p = jnp.exp(st)
        l_ref[i] = jnp.sum(p, axis=0)
        acc = lax.dot_general(v_ref[i], p.astype(jnp.bfloat16),
                              (((1,), (0,)), ((), ())),
                              preferred_element_type=jnp.float32)
        o_ref[i] = acc.astype(jnp.bfloat16)

    def remote_head(i):
        qt = q_ref[i]
        st = lax.dot_general(ko_ref[i], qt, (((0,), (0,)), ((), ())),
                             preferred_element_type=jnp.float32)
        p = jnp.exp(st)
        l = l_ref[i] + jnp.sum(p, axis=0)
        acc = (o_ref[i].astype(jnp.float32) +
               lax.dot_general(vo_ref[i], p.astype(jnp.bfloat16),
                               (((1,), (0,)), ((), ())),
                               preferred_element_type=jnp.float32))
        o_ref[i] = (acc / l[None, :]).astype(jnp.bfloat16)

    rdma_b = []
    for h in range(N_HALF):
        if compute:
            local_head(base_a + h)
            local_head(base_b + h)
        if comm:
            rk, rv = rdma_a[h]
            rk.wait()
            rv.wait()
            hd = base_a + h
            fk = pltpu.make_async_remote_copy(
                src_ref=ko_ref.at[hd], dst_ref=ko_ref.at[hd],
                send_sem=sb_sems.at[0, h], recv_sem=rb_sems.at[0, h],
                device_id=nbr_x, device_id_type=pl.DeviceIdType.MESH)
            fv = pltpu.make_async_remote_copy(
                src_ref=vo_ref.at[hd], dst_ref=vo_ref.at[hd],
                send_sem=sb_sems.at[1, h], recv_sem=rb_sems.at[1, h],
                device_id=nbr_x, device_id_type=pl.DeviceIdType.MESH)
            fk.start()
            fv.start()
            rdma_b.append((fk, fv))
        if compute:
            remote_head(base_a + h)

    for h in range(N_HALF):
        if comm:
            fk, fv = rdma_b[h]
            fk.wait()
            fv.wait()
        if compute:
            remote_head(base_b + h)

    if MODE in ("comm", "noexit"):
        o_ref[...] = jnp.zeros_like(o_ref)


def kernel(Q, K, V):
    b, s, h, d = Q.shape

    def prep(a):
        return a.astype(jnp.bfloat16).transpose(0, 2, 3, 1).reshape(b * h, d, s)

    out = pl.pallas_call(
        _body,
        out_shape=jax.ShapeDtypeStruct((b * h, d, s), jnp.bfloat16),
        in_specs=[pl.BlockSpec(memory_space=pltpu.VMEM)] * 3,
        out_specs=pl.BlockSpec(memory_space=pltpu.VMEM),
        scratch_shapes=[
            pltpu.VMEM((b * h, d, s), jnp.bfloat16),
            pltpu.VMEM((b * h, d, s), jnp.bfloat16),
            pltpu.VMEM((b * h, s), jnp.float32),
            pltpu.SemaphoreType.DMA((2, N_HALF)),
            pltpu.SemaphoreType.DMA((2, N_HALF)),
            pltpu.SemaphoreType.DMA((2, N_HALF)),
            pltpu.SemaphoreType.DMA((2, N_HALF)),
        ],
        compiler_params=(pltpu.CompilerParams() if MODE == "raw"
                         else pltpu.CompilerParams(collective_id=0)),
    )(prep(Q * SCALE), prep(K), prep(V))
    return out.reshape(b, h, d, s).transpose(0, 3, 1, 2)


# device time: 26140 ns/iter; 1.0239x vs baseline; 1.0005x over previous
import pathlib

import jax
import jax.numpy as jnp
from jax import lax
from jax.experimental import pallas as pl
from jax.experimental.pallas import tpu as pltpu

SCALE = 64 ** -0.5

_MODE_FILE = pathlib.Path(__file__).parent / "bench_mode.txt"
MODE = _MODE_FILE.read_text().strip() if _MODE_FILE.exists() else "full"

N_HALF = 8


def _body(q_ref, k_ref, v_ref, o_ref, ko_ref, vo_ref, l_ref,
          sa_sems, ra_sems, sb_sems, rb_sems):
    my_x = lax.axis_index("x")
    my_y = lax.axis_index("y")
    nbr_y = (my_x, 1 - my_y)
    nbr_x = (1 - my_x, my_y)

    comm = MODE in ("full", "comm")
    compute = MODE not in ("comm", "raw", "noexit")

    if MODE == "raw":
        o_ref[...] = jnp.zeros_like(o_ref)
        return

    barrier_sem = pltpu.get_barrier_semaphore()
    for nbr in (nbr_y, nbr_x):
        pl.semaphore_signal(barrier_sem, inc=1, device_id=nbr,
                            device_id_type=pl.DeviceIdType.MESH)
    pl.semaphore_wait(barrier_sem, 2)

    base_a = my_x * N_HALF
    base_b = (1 - my_x) * N_HALF

    if MODE in ("compute", "none"):
        ko_ref, vo_ref = k_ref, v_ref

    rdma_a = []
    if comm:
        for h in range(N_HALF):
            hd = base_a + h
            rk = pltpu.make_async_remote_copy(
                src_ref=k_ref.at[hd], dst_ref=ko_ref.at[hd],
                send_sem=sa_sems.at[0, h], recv_sem=ra_sems.at[0, h],
                device_id=nbr_y, device_id_type=pl.DeviceIdType.MESH)
            rv = pltpu.make_async_remote_copy(
                src_ref=v_ref.at[hd], dst_ref=vo_ref.at[hd],
                send_sem=sa_sems.at[1, h], recv_sem=ra_sems.at[1, h],
                device_id=nbr_y, device_id_type=pl.DeviceIdType.MESH)
            rk.start()
            rv.start()
            rdma_a.append((rk, rv))

    def local_head(i):
        qt = q_ref[i]
        st = lax.dot_general(k_ref[i], qt, (((0,), (0,)), ((), ())),
                             preferred_element_type=jnp.float32)
        p = jnp.exp(st)
        l_ref[i] = jnp.sum(p, axis=0)
        acc = lax.dot_general(v_ref[i], p.astype(jnp.bfloat16),
                              (((1,), (0,)), ((), ())),
                              preferred_element_type=jnp.float32)
        o_ref[i] = acc.astype(jnp.bfloat16)

    def remote_head(i):
        qt = q_ref[i]
        st = lax.dot_general(ko_ref[i], qt, (((0,), (0,)), ((), ())),
                             preferred_element_type=jnp.float32)
        p = jnp.exp(st)
        l = l_ref[i] + jnp.sum(p, axis=0)
        acc = (o_ref[i].astype(jnp.float32) +
               lax.dot_general(vo_ref[i], p.astype(jnp.bfloat16),
                               (((1,), (0,)), ((), ())),
                               preferred_element_type=jnp.float32))
        o_ref[i] = (acc / l[None, :]).astype(jnp.bfloat16)

    if compute:
        local_head(base_a)
        local_head(base_b)
    rdma_b = []
    for h in range(N_HALF):
        if comm:
            rk, rv = rdma_a[h]
            hd = base_a + h
            rk.wait()
            fk = pltpu.make_async_remote_copy(
                src_ref=ko_ref.at[hd], dst_ref=ko_ref.at[hd],
                send_sem=sb_sems.at[0, h], recv_sem=rb_sems.at[0, h],
                device_id=nbr_x, device_id_type=pl.DeviceIdType.MESH)
            fk.start()
            rv.wait()
            fv = pltpu.make_async_remote_copy(
                src_ref=vo_ref.at[hd], dst_ref=vo_ref.at[hd],
                send_sem=sb_sems.at[1, h], recv_sem=rb_sems.at[1, h],
                device_id=nbr_x, device_id_type=pl.DeviceIdType.MESH)
            fv.start()
            rdma_b.append((fk, fv))
        if compute:
            remote_head(base_a + h)
            if h + 1 < N_HALF:
                local_head(base_a + h + 1)
                local_head(base_b + h + 1)

    for h in range(N_HALF):
        if comm:
            fk, fv = rdma_b[h]
            fk.wait()
            fv.wait()
        if compute:
            remote_head(base_b + h)

    if MODE in ("comm", "noexit"):
        o_ref[...] = jnp.zeros_like(o_ref)


def kernel(Q, K, V):
    b, s, h, d = Q.shape

    def prep(a):
        return a.astype(jnp.bfloat16).transpose(0, 2, 3, 1).reshape(b * h, d, s)

    out = pl.pallas_call(
        _body,
        out_shape=jax.ShapeDtypeStruct((b * h, d, s), jnp.bfloat16),
        in_specs=[pl.BlockSpec(memory_space=pltpu.VMEM)] * 3,
        out_specs=pl.BlockSpec(memory_space=pltpu.VMEM),
        scratch_shapes=[
            pltpu.VMEM((b * h, d, s), jnp.bfloat16),
            pltpu.VMEM((b * h, d, s), jnp.bfloat16),
            pltpu.VMEM((b * h, s), jnp.float32),
            pltpu.SemaphoreType.DMA((2, N_HALF)),
            pltpu.SemaphoreType.DMA((2, N_HALF)),
            pltpu.SemaphoreType.DMA((2, N_HALF)),
            pltpu.SemaphoreType.DMA((2, N_HALF)),
        ],
        compiler_params=(pltpu.CompilerParams() if MODE == "raw"
                         else pltpu.CompilerParams(collective_id=0)),
    )(prep(Q * SCALE), prep(K), prep(V))
    return out.reshape(b, h, d, s).transpose(0, 3, 1, 2)
